# baseline (speedup 1.0000x reference)
import functools

import jax
import jax.numpy as jnp
from jax import lax
from jax.experimental import pallas as pl
from jax.experimental.pallas import tpu as pltpu
from jax.experimental.pallas import tpu_sc as plsc

_MAX_STEPS = 1000
_BATCH = 16384
_E = 128
_D = 512

_NC = 2
_NS = 16
_NW = _NC * _NS
_ROWS_PER_W = _BATCH // _NW
_CHUNK = 128
_K = _ROWS_PER_W // _CHUNK

_MLP_BLK = 1024


def _build_table():
    steps = jnp.arange(_MAX_STEPS, dtype=jnp.float32)[:, None]
    dims = jnp.arange(64, dtype=jnp.float32)[None, :]
    t = steps * 10.0 ** (dims * 4.0 / 63.0)
    return jnp.concatenate([jnp.sin(t), jnp.cos(t)], axis=1)


def _gather_body(table_hbm, idx_hbm, out_hbm, idx_v,
                 rows0, rows1, gsem0, gsem1, wsem0, wsem1):
    wid = lax.axis_index("s") * _NC + lax.axis_index("c")
    base = wid * _ROWS_PER_W
    pltpu.sync_copy(idx_hbm.at[pl.ds(base, _ROWS_PER_W)], idx_v)

    def gath(c, buf, sem):
        return pltpu.async_copy(
            table_hbm.at[idx_v.at[pl.ds(c * _CHUNK, _CHUNK)]], buf, sem)

    def wr(c, buf, sem):
        return pltpu.async_copy(
            buf, out_hbm.at[pl.ds(base + c * _CHUNK, _CHUNK)], sem)

    bufs = (rows0, rows1)
    gsems = (gsem0, gsem1)
    wsems = (wsem0, wsem1)
    g = [None, None]
    w = [None, None]
    for j in range(_K):
        b = j % 2
        if w[b] is not None:
            w[b].wait()
        g[b] = gath(j, bufs[b], gsems[b])
        if j >= 1:
            bb = (j - 1) % 2
            g[bb].wait()
            w[bb] = wr(j - 1, bufs[bb], wsems[bb])
    g[(_K - 1) % 2].wait()
    w[(_K - 1) % 2] = wr(_K - 1, bufs[(_K - 1) % 2], wsems[(_K - 1) % 2])
    w[0].wait()
    w[1].wait()


def _sc_gather(table, idx):
    mesh = plsc.VectorSubcoreMesh(core_axis_name="c", subcore_axis_name="s")
    k = functools.partial(
        pl.kernel,
        mesh=mesh,
        out_type=jax.ShapeDtypeStruct((_BATCH, _E), jnp.float32),
        scratch_types=[
            pltpu.VMEM((_ROWS_PER_W,), jnp.int32),
            pltpu.VMEM((_CHUNK, _E), jnp.float32),
            pltpu.VMEM((_CHUNK, _E), jnp.float32),
            pltpu.SemaphoreType.DMA,
            pltpu.SemaphoreType.DMA,
            pltpu.SemaphoreType.DMA,
            pltpu.SemaphoreType.DMA,
        ],
    )(_gather_body)
    return k(table, idx)


def _mlp_body(x_ref, w1_ref, b1_ref, w2_ref, b2_ref, o_ref):
    x = x_ref[...].astype(jnp.bfloat16)
    h = jnp.dot(x, w1_ref[...], preferred_element_type=jnp.float32) + b1_ref[...]
    h = h * jax.nn.sigmoid(h)
    o = jnp.dot(h.astype(jnp.bfloat16), w2_ref[...],
                preferred_element_type=jnp.float32) + b2_ref[...]
    o_ref[...] = o * jax.nn.sigmoid(o)


def _tc_mlp(x, W1, b1, W2, b2):
    nb = _BATCH // _MLP_BLK
    return pl.pallas_call(
        _mlp_body,
        grid=(nb,),
        in_specs=[
            pl.BlockSpec((_MLP_BLK, _E), lambda i: (i, 0)),
            pl.BlockSpec((_E, _D), lambda i: (0, 0)),
            pl.BlockSpec((_D,), lambda i: (0,)),
            pl.BlockSpec((_D, _D), lambda i: (0, 0)),
            pl.BlockSpec((_D,), lambda i: (0,)),
        ],
        out_specs=pl.BlockSpec((_MLP_BLK, _D), lambda i: (i, 0)),
        out_shape=jax.ShapeDtypeStruct((_BATCH, _D), jnp.float32),
    )(x, W1.astype(jnp.bfloat16), b1, W2.astype(jnp.bfloat16), b2)


def kernel(diffusion_step, W1, b1, W2, b2):
    table = _build_table()
    x = _sc_gather(table, diffusion_step.astype(jnp.int32))
    return _tc_mlp(x, W1, b1, W2, b2)


# speedup vs baseline: 1.1712x; 1.1712x over previous
import functools

import jax
import jax.numpy as jnp
from jax import lax
from jax.experimental import pallas as pl
from jax.experimental.pallas import tpu as pltpu
from jax.experimental.pallas import tpu_sc as plsc

_MAX_STEPS = 1000
_BATCH = 16384
_D = 512
_TPAD = 1024

_NC = 2
_NS = 16
_NW = _NC * _NS
_ROWS_PER_W = _BATCH // _NW
_CHUNK = 64
_K = _ROWS_PER_W // _CHUNK


def _mlp_body(w1_ref, b1_ref, w2_ref, b2_ref, o_ref):
    steps = lax.broadcasted_iota(jnp.int32, (_TPAD, 64), 0).astype(jnp.float32)
    dims = lax.broadcasted_iota(jnp.int32, (_TPAD, 64), 1).astype(jnp.float32)
    t = steps * 10.0 ** (dims * 4.0 / 63.0)
    x = jnp.concatenate([jnp.sin(t), jnp.cos(t)], axis=1)
    h = jnp.dot(x, w1_ref[...], preferred_element_type=jnp.float32) + b1_ref[...]
    h = h * jax.nn.sigmoid(h)
    o = jnp.dot(h, w2_ref[...], preferred_element_type=jnp.float32) + b2_ref[...]
    o_ref[...] = o * jax.nn.sigmoid(o)


def _tc_mlp(W1, b1, W2, b2):
    return pl.pallas_call(
        _mlp_body,
        out_shape=jax.ShapeDtypeStruct((_TPAD, _D), jnp.float32),
    )(W1, b1, W2, b2)


def _gather_body(table_hbm, idx_hbm, out_hbm, idx_v,
                 rows0, rows1, gsem0, gsem1, wsem0, wsem1):
    wid = lax.axis_index("s") * _NC + lax.axis_index("c")
    base = wid * _ROWS_PER_W
    pltpu.sync_copy(idx_hbm.at[pl.ds(base, _ROWS_PER_W)], idx_v)

    def gath(c, buf, sem):
        return pltpu.async_copy(
            table_hbm.at[idx_v.at[pl.ds(c * _CHUNK, _CHUNK)]], buf, sem)

    def wr(c, buf, sem):
        return pltpu.async_copy(
            buf, out_hbm.at[pl.ds(base + c * _CHUNK, _CHUNK)], sem)

    bufs = (rows0, rows1)
    gsems = (gsem0, gsem1)
    wsems = (wsem0, wsem1)
    g = [None, None]
    w = [None, None]
    for j in range(_K):
        b = j % 2
        if w[b] is not None:
            w[b].wait()
        g[b] = gath(j, bufs[b], gsems[b])
        if j >= 1:
            bb = (j - 1) % 2
            g[bb].wait()
            w[bb] = wr(j - 1, bufs[bb], wsems[bb])
    g[(_K - 1) % 2].wait()
    w[(_K - 1) % 2] = wr(_K - 1, bufs[(_K - 1) % 2], wsems[(_K - 1) % 2])
    w[0].wait()
    w[1].wait()


def _sc_gather(final_table, idx):
    mesh = plsc.VectorSubcoreMesh(core_axis_name="c", subcore_axis_name="s")
    k = functools.partial(
        pl.kernel,
        mesh=mesh,
        out_type=jax.ShapeDtypeStruct((_BATCH, _D), jnp.float32),
        scratch_types=[
            pltpu.VMEM((_ROWS_PER_W,), jnp.int32),
            pltpu.VMEM((_CHUNK, _D), jnp.float32),
            pltpu.VMEM((_CHUNK, _D), jnp.float32),
            pltpu.SemaphoreType.DMA,
            pltpu.SemaphoreType.DMA,
            pltpu.SemaphoreType.DMA,
            pltpu.SemaphoreType.DMA,
        ],
    )(_gather_body)
    return k(final_table, idx)


def kernel(diffusion_step, W1, b1, W2, b2):
    final_table = _tc_mlp(W1, b1, W2, b2)
    return _sc_gather(final_table, diffusion_step.astype(jnp.int32))
